# Initial kernel scaffold; baseline (speedup 1.0000x reference)
import jax
import jax.numpy as jnp
from jax.experimental import pallas as pl


def kernel(x, edge_attr, Wm, bm, Wih, Whh, bih, bhh, edge_index):
    raise NotImplementedError("write your pallas kernel here")


# speedup vs baseline: 6.2444x; 6.2444x over previous
import functools

import jax
import jax.numpy as jnp
from jax import lax
from jax.experimental import pallas as pl
from jax.experimental.pallas import tpu as pltpu
from jax.experimental.pallas import tpu_sc as plsc

_N = 10000
_E = 320000
_D = 128
_NC = 2
_NS = 16
_NW = _NC * _NS
_PER_W = _E // _NW
_K = 80
_NG = 5
_GC = 25
_NCHUNK = _PER_W // _K
_NP = 10240
_RPT = _NP // _NS
_ZR = 40
_ZITER = _RPT // _ZR

_f32 = jnp.float32


def _zero_vmem(ref, rows, cols):
    @pl.loop(0, rows)
    def _(i):
        for c in range(cols // 16):
            ref[i, pl.ds(c * 16, 16)] = jnp.zeros((16,), _f32)


def _sc_mesh():
    return plsc.VectorSubcoreMesh(core_axis_name="c", subcore_axis_name="s")


def _sc_segsum(h, src4, dstf):
    @functools.partial(
        pl.kernel,
        out_type=jax.ShapeDtypeStruct((_NC, _NP, _D), _f32),
        mesh=_sc_mesh(),
        scratch_types=[
            pltpu.VMEM((_GC, _K), jnp.int32),
            pltpu.VMEM((_K,), jnp.int32),
            pltpu.VMEM((_K, _D), _f32),
            pltpu.VMEM((_ZR, _D), _f32),
            pltpu.VMEM_SHARED((_NP, _D), _f32),
        ],
    )
    def k(h_hbm, src_hbm, dst_hbm, out_hbm, src_v, dst_v, buf, zbuf, acc):
        cid = lax.axis_index("c")
        sid = lax.axis_index("s")
        wid = sid * _NC + cid
        _zero_vmem(zbuf, _ZR, _D)

        @pl.loop(0, _ZITER)
        def _(i):
            pltpu.sync_copy(zbuf, acc.at[pl.ds(sid * _RPT + i * _ZR, _ZR)])

        plsc.subcore_barrier()

        @pl.loop(0, _NG)
        def _(g):
            pltpu.sync_copy(src_hbm.at[wid, g], src_v)

            @pl.loop(0, _GC)
            def _(j):
                b = wid * _PER_W + g * (_GC * _K) + j * _K
                pltpu.sync_copy(dst_hbm.at[pl.ds(b, _K)], dst_v)
                pltpu.sync_copy(h_hbm.at[src_v.at[j]], buf)
                pltpu.sync_copy(buf, acc.at[dst_v], add=True)

        plsc.subcore_barrier()

        @pl.loop(0, _ZITER)
        def _(i):
            r0 = sid * _RPT + i * _ZR
            pltpu.sync_copy(acc.at[pl.ds(r0, _ZR)], zbuf)
            pltpu.sync_copy(zbuf, out_hbm.at[cid, pl.ds(r0, _ZR)])

    return k(h, src4, dstf)


def _sc_pre(edge_attr, dstf):
    @functools.partial(
        pl.kernel,
        out_type=(jax.ShapeDtypeStruct((_NC, _NP, _D), _f32),
                  jax.ShapeDtypeStruct((_NC, _NP, _D), _f32)),
        mesh=_sc_mesh(),
        scratch_types=[
            pltpu.VMEM((_K,), jnp.int32),
            pltpu.VMEM((_K, _D), _f32),
            pltpu.VMEM((_K, _D), _f32),
            pltpu.VMEM((_ZR, _D), _f32),
            pltpu.VMEM_SHARED((_NP, _D), _f32),
        ],
    )
    def k(ea_hbm, dst_hbm, oute_hbm, outd_hbm,
          dst_v, eav, ones_v, zbuf, acc):
        cid = lax.axis_index("c")
        sid = lax.axis_index("s")
        wid = sid * _NC + cid
        _zero_vmem(zbuf, _ZR, _D)

        @pl.loop(0, _K)
        def _(i):
            for c in range(_D // 16):
                ones_v[i, pl.ds(c * 16, 16)] = jnp.ones((16,), _f32)

        @pl.loop(0, _ZITER)
        def _(i):
            pltpu.sync_copy(zbuf, acc.at[pl.ds(sid * _RPT + i * _ZR, _ZR)])

        plsc.subcore_barrier()

        @pl.loop(0, _NCHUNK)
        def _(j):
            b = wid * _PER_W + j * _K
            pltpu.sync_copy(dst_hbm.at[pl.ds(b, _K)], dst_v)
            pltpu.sync_copy(ea_hbm.at[pl.ds(b, _K)], eav)
            pltpu.sync_copy(eav, acc.at[dst_v], add=True)

        plsc.subcore_barrier()

        @pl.loop(0, _ZITER)
        def _(i):
            r0 = sid * _RPT + i * _ZR
            pltpu.sync_copy(acc.at[pl.ds(r0, _ZR)], zbuf)
            pltpu.sync_copy(zbuf, oute_hbm.at[cid, pl.ds(r0, _ZR)])

        _zero_vmem(zbuf, _ZR, _D)

        @pl.loop(0, _ZITER)
        def _(i):
            pltpu.sync_copy(zbuf, acc.at[pl.ds(sid * _RPT + i * _ZR, _ZR)])

        plsc.subcore_barrier()

        @pl.loop(0, _NCHUNK)
        def _(j):
            b = wid * _PER_W + j * _K
            pltpu.sync_copy(dst_hbm.at[pl.ds(b, _K)], dst_v)
            pltpu.sync_copy(ones_v, acc.at[dst_v], add=True)

        plsc.subcore_barrier()

        @pl.loop(0, _ZITER)
        def _(i):
            r0 = sid * _RPT + i * _ZR
            pltpu.sync_copy(acc.at[pl.ds(r0, _ZR)], zbuf)
            pltpu.sync_copy(zbuf, outd_hbm.at[cid, pl.ds(r0, _ZR)])

    return k(edge_attr, dstf)


def _tc_round(h, s0, s1, e0, e1, d0, d1, wv, wu, we, bmv, wih, whh, bihv, bhhv):
    bn = 1000
    grid = (_N // bn,)
    hi = lax.Precision.HIGHEST

    def body(h_ref, s0r, s1r, e0r, e1r, d0r, d1r,
             wvr, wur, wer, bmr, wihr, whhr, bihr, bhhr, out_ref):
        hb = h_ref[...]
        s = s0r[...] + s1r[...]
        eg = e0r[...] + e1r[...]
        deg = d0r[...][:, :1] + d1r[...][:, :1]
        pos = deg > 0.0
        inv = jnp.where(pos, 1.0 / jnp.maximum(deg, 1.0), 0.0)
        base = jnp.dot(hb, wvr[...], precision=hi,
                       preferred_element_type=_f32) + bmr[...]
        agg = (jnp.dot(s, wur[...], precision=hi, preferred_element_type=_f32)
               + jnp.dot(eg, wer[...], precision=hi,
                         preferred_element_type=_f32))
        act = jnp.where(pos, base, 0.0) + inv * agg
        gi = jnp.dot(act, wihr[...], precision=hi,
                     preferred_element_type=_f32) + bihr[...]
        gh = jnp.dot(hb, whhr[...], precision=hi,
                     preferred_element_type=_f32) + bhhr[...]
        r = jax.nn.sigmoid(gi[:, :_D] + gh[:, :_D])
        z = jax.nn.sigmoid(gi[:, _D:2 * _D] + gh[:, _D:2 * _D])
        n = jnp.tanh(gi[:, 2 * _D:] + r * gh[:, 2 * _D:])
        out_ref[...] = (1.0 - z) * n + z * hb

    row_spec = pl.BlockSpec((bn, _D), lambda i: (i, 0))
    deg_spec = row_spec

    def full(a):
        return pl.BlockSpec(a.shape, lambda i: tuple(0 for _ in a.shape))

    return pl.pallas_call(
        body,
        grid=grid,
        in_specs=[row_spec, row_spec, row_spec, row_spec, row_spec,
                  deg_spec, deg_spec,
                  full(wv), full(wu), full(we), full(bmv),
                  full(wih), full(whh), full(bihv), full(bhhv)],
        out_specs=row_spec,
        out_shape=jax.ShapeDtypeStruct((_N, _D), _f32),
    )(h, s0, s1, e0, e1, d0, d1, wv, wu, we, bmv, wih, whh, bihv, bhhv)


def kernel(x, edge_attr, Wm, bm, Wih, Whh, bih, bhh, edge_index):
    ei = edge_index.astype(jnp.int32)
    src4 = ei[0].reshape(_NW, _NG, _GC, _K)
    dstf = ei[1]

    eagg, deg = _sc_pre(edge_attr, dstf)

    h = x
    for t in range(2):
        sp = _sc_segsum(h, src4, dstf)
        wv = Wm[t, :, :_D].T
        wu = Wm[t, :, _D:2 * _D].T
        we = Wm[t, :, 2 * _D:].T
        h = _tc_round(h, sp[0], sp[1], eagg[0], eagg[1], deg[0], deg[1],
                      wv, wu, we, bm[t][None],
                      Wih[t].T, Whh[t].T, bih[t][None], bhh[t][None])
    return h


# speedup vs baseline: 11.0129x; 1.7637x over previous
import dataclasses
import functools

import jax
import jax.numpy as jnp
from jax import lax
from jax.experimental import pallas as pl
from jax.experimental.pallas import tpu as pltpu
from jax.experimental.pallas import tpu_sc as plsc

_N = 10000
_E = 320000
_D = 128
_NC = 2
_NS = 16
_NW = _NC * _NS
_PER_W = _E // _NW
_K = 80
_NG = 5
_GC = 25
_NCHUNK = _PER_W // _K
_NP = 10240
_RPT = _NP // _NS
_ZR = 32
_ZITER = _RPT // _ZR

_f32 = jnp.float32

_SC_CP = pltpu.CompilerParams()
if "needs_layout_passes" in pltpu.CompilerParams.__dataclass_fields__:
    _SC_CP = dataclasses.replace(_SC_CP, needs_layout_passes=False)


def _fill_vmem(ref, rows, cols, value):
    @pl.loop(0, rows)
    def _(i):
        for c in range(cols // 16):
            ref[i, pl.ds(c * 16, 16)] = jnp.full((16,), value, _f32)


def _sc_mesh():
    return plsc.VectorSubcoreMesh(core_axis_name="c", subcore_axis_name="s")


def _zero_acc(acc, zbuf, sid):
    @pl.loop(0, _ZITER)
    def _(i):
        pltpu.sync_copy(zbuf, acc.at[pl.ds(sid * _RPT + i * _ZR, _ZR)])


def _copy_out(acc, zbuf, out_hbm, cid, sid):
    @pl.loop(0, _ZITER)
    def _(i):
        r0 = sid * _RPT + i * _ZR
        pltpu.sync_copy(acc.at[pl.ds(r0, _ZR)], zbuf)
        pltpu.sync_copy(zbuf, out_hbm.at[cid, pl.ds(r0, _ZR)])


def _sc_segsum(h, src4, dstf):
    @functools.partial(
        pl.kernel,
        out_type=jax.ShapeDtypeStruct((_NC, _NP, _D), _f32),
        mesh=_sc_mesh(),
        scratch_types=[
            pltpu.VMEM((_GC, _K), jnp.int32),
            pltpu.VMEM((_K,), jnp.int32),
            pltpu.VMEM((_K,), jnp.int32),
            pltpu.VMEM((_K, _D), _f32),
            pltpu.VMEM((_K, _D), _f32),
            pltpu.VMEM((_ZR, _D), _f32),
            pltpu.VMEM_SHARED((_NP, _D), _f32),
            pltpu.SemaphoreType.DMA,
            pltpu.SemaphoreType.DMA,
            pltpu.SemaphoreType.DMA,
            pltpu.SemaphoreType.DMA,
        ],
    )
    def k(h_hbm, src_hbm, dst_hbm, out_hbm, src_v, dsta, dstb, bufa, bufb,
          zbuf, acc, gsa, gsb, dsa, dsb):
        cid = lax.axis_index("c")
        sid = lax.axis_index("s")
        wid = sid * _NC + cid
        _fill_vmem(zbuf, _ZR, _D, 0.0)
        _zero_acc(acc, zbuf, sid)
        plsc.subcore_barrier()

        @pl.loop(0, _NG)
        def _(g):
            pltpu.sync_copy(src_hbm.at[wid, g], src_v)
            base = wid * _PER_W + g * (_GC * _K)

            def issue(j, buf, dst, gs, ds):
                pltpu.async_copy(h_hbm.at[src_v.at[j]], buf, gs)
                pltpu.async_copy(dst_hbm.at[pl.ds(base + j * _K, _K)],
                                 dst, ds)

            def finish(j, buf, dst, gs, ds):
                pltpu.make_async_copy(h_hbm.at[src_v.at[j]], buf, gs).wait()
                pltpu.make_async_copy(dst_hbm.at[pl.ds(base + j * _K, _K)],
                                      dst, ds).wait()
                pltpu.sync_copy(buf, acc.at[dst], add=True)

            issue(0, bufa, dsta, gsa, dsa)

            @pl.loop(0, (_GC - 1) // 2)
            def _(p):
                issue(1 + 2 * p, bufb, dstb, gsb, dsb)
                finish(2 * p, bufa, dsta, gsa, dsa)
                issue(2 + 2 * p, bufa, dsta, gsa, dsa)
                finish(1 + 2 * p, bufb, dstb, gsb, dsb)

            finish(_GC - 1, bufa, dsta, gsa, dsa)

        plsc.subcore_barrier()
        _copy_out(acc, zbuf, out_hbm, cid, sid)

    return k(h, src4, dstf)


def _sc_pre_seg0(x, edge_attr, src4, dstf):
    @functools.partial(
        pl.kernel,
        out_type=(jax.ShapeDtypeStruct((_NC, _NP, _D), _f32),
                  jax.ShapeDtypeStruct((_NC, _NP, _D), _f32)),
        mesh=_sc_mesh(),
        scratch_types=[
            pltpu.VMEM((_GC, _K), jnp.int32),
            pltpu.VMEM((_K,), jnp.int32),
            pltpu.VMEM((_K,), jnp.int32),
            pltpu.VMEM((_K, _D), _f32),
            pltpu.VMEM((_K, _D), _f32),
            pltpu.VMEM((_ZR, _D), _f32),
            pltpu.VMEM_SHARED((_NP, _D), _f32),
            pltpu.SemaphoreType.DMA,
            pltpu.SemaphoreType.DMA,
            pltpu.SemaphoreType.DMA,
            pltpu.SemaphoreType.DMA,
        ],
    )
    def k(h_hbm, ea_hbm, src_hbm, dst_hbm, outs_hbm, oute_hbm,
          src_v, dsta, dstb, bufa, bufb, zbuf, acc, gsa, gsb, dsa, dsb):
        cid = lax.axis_index("c")
        sid = lax.axis_index("s")
        wid = sid * _NC + cid
        base = wid * _PER_W
        _fill_vmem(zbuf, _ZR, _D, 0.0)
        _zero_acc(acc, zbuf, sid)
        plsc.subcore_barrier()

        @pl.loop(0, _NG)
        def _(g):
            pltpu.sync_copy(src_hbm.at[wid, g], src_v)
            gbase = base + g * (_GC * _K)

            def issue(j, buf, dst, gs, ds):
                pltpu.async_copy(h_hbm.at[src_v.at[j]], buf, gs)
                pltpu.async_copy(dst_hbm.at[pl.ds(gbase + j * _K, _K)],
                                 dst, ds)

            def finish(j, buf, dst, gs, ds):
                pltpu.make_async_copy(h_hbm.at[src_v.at[j]], buf, gs).wait()
                pltpu.make_async_copy(dst_hbm.at[pl.ds(gbase + j * _K, _K)],
                                      dst, ds).wait()
                pltpu.sync_copy(buf, acc.at[dst], add=True)

            issue(0, bufa, dsta, gsa, dsa)

            @pl.loop(0, (_GC - 1) // 2)
            def _(p):
                issue(1 + 2 * p, bufb, dstb, gsb, dsb)
                finish(2 * p, bufa, dsta, gsa, dsa)
                issue(2 + 2 * p, bufa, dsta, gsa, dsa)
                finish(1 + 2 * p, bufb, dstb, gsb, dsb)

            finish(_GC - 1, bufa, dsta, gsa, dsa)

        plsc.subcore_barrier()
        _copy_out(acc, zbuf, outs_hbm, cid, sid)
        _fill_vmem(zbuf, _ZR, _D, 0.0)
        _zero_acc(acc, zbuf, sid)
        plsc.subcore_barrier()

        def issue_a(j, buf, dst, gs, ds):
            pltpu.async_copy(ea_hbm.at[pl.ds(base + j * _K, _K)], buf, gs)
            pltpu.async_copy(dst_hbm.at[pl.ds(base + j * _K, _K)], dst, ds)

        def finish_a(j, buf, dst, gs, ds):
            pltpu.make_async_copy(ea_hbm.at[pl.ds(base + j * _K, _K)],
                                  buf, gs).wait()
            pltpu.make_async_copy(dst_hbm.at[pl.ds(base + j * _K, _K)],
                                  dst, ds).wait()
            pltpu.sync_copy(buf, acc.at[dst], add=True)

        issue_a(0, bufa, dsta, gsa, dsa)

        @pl.loop(0, (_NCHUNK - 1) // 2)
        def _(p):
            issue_a(1 + 2 * p, bufb, dstb, gsb, dsb)
            finish_a(2 * p, bufa, dsta, gsa, dsa)
            issue_a(2 + 2 * p, bufa, dsta, gsa, dsa)
            finish_a(1 + 2 * p, bufb, dstb, gsb, dsb)

        finish_a(_NCHUNK - 1, bufa, dsta, gsa, dsa)

        plsc.subcore_barrier()
        _copy_out(acc, zbuf, oute_hbm, cid, sid)

    return k(x, edge_attr, src4, dstf)


def _sc_deg(dstf):
    @functools.partial(
        pl.kernel,
        out_type=(jax.ShapeDtypeStruct((_NC, _NP * _D), _f32),
                  jax.ShapeDtypeStruct((_NW, _NP), _f32)),
        mesh=_sc_mesh(),
        compiler_params=_SC_CP,
        scratch_types=[
            pltpu.VMEM((_PER_W,), jnp.int32),
            pltpu.VMEM((_NP,), _f32),
            pltpu.VMEM((_RPT,), _f32),
            pltpu.VMEM((_ZR * _D,), _f32),
        ],
    )
    def k(dst_hbm, outd_hbm, outh_hbm, dstv, hist, hred, stage):
        cid = lax.axis_index("c")
        sid = lax.axis_index("s")
        wid = sid * _NC + cid
        pltpu.sync_copy(dst_hbm.at[pl.ds(wid * _PER_W, _PER_W)], dstv)

        @pl.loop(0, _NP // 16)
        def _(i):
            hist[pl.ds(i * 16, 16)] = jnp.zeros((16,), _f32)

        ones16 = jnp.ones((16,), _f32)

        @pl.loop(0, _PER_W // 16)
        def _(i):
            plsc.addupdate_scatter(hist, [dstv[pl.ds(i * 16, 16)]], ones16)

        pltpu.sync_copy(hist, outh_hbm.at[wid])
        plsc.subcore_barrier()

        @pl.loop(0, _RPT // 16)
        def _(i):
            hred[pl.ds(i * 16, 16)] = jnp.zeros((16,), _f32)

        @pl.loop(0, _NS)
        def _(w):
            pltpu.sync_copy(
                outh_hbm.at[w * _NC + cid, pl.ds(sid * _RPT, _RPT)],
                hist.at[pl.ds(0, _RPT)])

            @pl.loop(0, _RPT // 16)
            def _(i):
                hred[pl.ds(i * 16, 16)] = (hred[pl.ds(i * 16, 16)]
                                           + hist[pl.ds(i * 16, 16)])

        @pl.loop(0, _ZITER)
        def _(i):

            @pl.loop(0, _ZR // 16)
            def _(q):
                v16 = hred[pl.ds(i * _ZR + q * 16, 16)]
                for r in range(16):
                    splat = lax.gather(
                        v16, jnp.full((16, 1), r, jnp.int32),
                        lax.GatherDimensionNumbers(
                            offset_dims=(), collapsed_slice_dims=(0,),
                            start_index_map=(0,)),
                        (1,), mode=lax.GatherScatterMode.PROMISE_IN_BOUNDS)
                    for c in range(_D // 16):
                        stage[pl.ds((q * 16 + r) * _D + c * 16, 16)] = splat

            pltpu.sync_copy(
                stage,
                outd_hbm.at[cid, pl.ds((sid * _RPT + i * _ZR) * _D,
                                       _ZR * _D)])

    return k(dstf)


def _tc_round(h, s0, s1, e0, e1, d0, d1, wv, wu, we, bmv, wih, whh, bihv, bhhv):
    bn = 1000
    grid = (_N // bn,)
    hi = lax.Precision.HIGHEST

    def body(h_ref, s0r, s1r, e0r, e1r, d0r, d1r,
             wvr, wur, wer, bmr, wihr, whhr, bihr, bhhr, out_ref):
        hb = h_ref[...]
        s = s0r[...] + s1r[...]
        eg = e0r[...] + e1r[...]
        deg = d0r[...][:, :1] + d1r[...][:, :1]
        pos = deg > 0.0
        inv = jnp.where(pos, 1.0 / jnp.maximum(deg, 1.0), 0.0)
        base = jnp.dot(hb, wvr[...], precision=hi,
                       preferred_element_type=_f32) + bmr[...]
        agg = (jnp.dot(s, wur[...], precision=hi, preferred_element_type=_f32)
               + jnp.dot(eg, wer[...], precision=hi,
                         preferred_element_type=_f32))
        act = jnp.where(pos, base, 0.0) + inv * agg
        gi = jnp.dot(act, wihr[...], precision=hi,
                     preferred_element_type=_f32) + bihr[...]
        gh = jnp.dot(hb, whhr[...], precision=hi,
                     preferred_element_type=_f32) + bhhr[...]
        r = jax.nn.sigmoid(gi[:, :_D] + gh[:, :_D])
        z = jax.nn.sigmoid(gi[:, _D:2 * _D] + gh[:, _D:2 * _D])
        n = jnp.tanh(gi[:, 2 * _D:] + r * gh[:, 2 * _D:])
        out_ref[...] = (1.0 - z) * n + z * hb

    row_spec = pl.BlockSpec((bn, _D), lambda i: (i, 0))

    def full(a):
        return pl.BlockSpec(a.shape, lambda i: tuple(0 for _ in a.shape))

    return pl.pallas_call(
        body,
        grid=grid,
        in_specs=[row_spec, row_spec, row_spec, row_spec, row_spec,
                  row_spec, row_spec,
                  full(wv), full(wu), full(we), full(bmv),
                  full(wih), full(whh), full(bihv), full(bhhv)],
        out_specs=row_spec,
        out_shape=jax.ShapeDtypeStruct((_N, _D), _f32),
    )(h, s0, s1, e0, e1, d0, d1, wv, wu, we, bmv, wih, whh, bihv, bhhv)


def kernel(x, edge_attr, Wm, bm, Wih, Whh, bih, bhh, edge_index):
    ei = edge_index.astype(jnp.int32)
    src4 = ei[0].reshape(_NW, _NG, _GC, _K)
    dstf = ei[1]

    s0, eagg = _sc_pre_seg0(x, edge_attr, src4, dstf)
    degf, _hist = _sc_deg(dstf)
    deg = degf.reshape(_NC, _NP, _D)

    h = x
    for t in range(2):
        sp = s0 if t == 0 else _sc_segsum(h, src4, dstf)
        wv = Wm[t, :, :_D].T
        wu = Wm[t, :, _D:2 * _D].T
        we = Wm[t, :, 2 * _D:].T
        h = _tc_round(h, sp[0], sp[1], eagg[0], eagg[1], deg[0], deg[1],
                      wv, wu, we, bm[t][None],
                      Wih[t].T, Whh[t].T, bih[t][None], bhh[t][None])
    return h
